# baseline (speedup 1.0000x reference)
import jax
import jax.numpy as jnp
from jax.experimental import pallas as pl
from jax.experimental.pallas import tpu as pltpu

_K = 144


def _bigru_kernel(xs_ref, ms_ref, w_ref, wof_ref, wob_ref, bout_ref,
                  out_ref, hx_ref, pf_ref, pb_ref):
    S = xs_ref.shape[0]
    H = wof_ref.shape[0]
    RT = xs_ref.shape[1]

    wof = wof_ref[:, :]
    wob = wob_ref[:, :]
    w = w_ref[:, :]
    half = jnp.bfloat16(0.5)

    hx_ref[:, :] = jnp.zeros_like(hx_ref)
    ones_pad = jnp.concatenate(
        [jnp.ones((1, RT), jnp.float32), jnp.zeros((1, RT), jnp.float32)],
        axis=0)
    hx_ref[pl.ds(2 * H + 2, 2), :] = ones_pad.astype(jnp.bfloat16)

    def step(t, carry):
        tb = S - 1 - t
        xf = xs_ref[pl.ds(t, 1), :]
        xb = xs_ref[pl.ds(tb, 1), :]
        hx_ref[pl.ds(2 * H, 2), :] = jnp.concatenate(
            [xf, xb], axis=0).astype(jnp.bfloat16)
        gates = jnp.dot(w, hx_ref[:, :],
                        preferred_element_type=jnp.float32
                        ).astype(jnp.bfloat16)

        th_f = jnp.tanh(gates[0:2 * H, :])
        hhp_f = gates[2 * H:3 * H, :]
        cf = jnp.tanh(gates[3 * H:4 * H, :] + hhp_f
                      + hhp_f * th_f[H:2 * H, :])
        hf = hx_ref[pl.ds(0, H), :]
        hfn = half * (hf + cf + th_f[0:H, :] * (cf - hf))

        th_b = jnp.tanh(gates[4 * H:6 * H, :])
        hhp_b = gates[6 * H:7 * H, :]
        cb = jnp.tanh(gates[7 * H:8 * H, :] + hhp_b
                      + hhp_b * th_b[H:2 * H, :])
        hb = hx_ref[pl.ds(H, H), :]
        hbn = half * (hb + cb + th_b[0:H, :] * (cb - hb))

        hx_ref[pl.ds(0, 2 * H), :] = jnp.concatenate([hfn, hbn], axis=0)

        pf_ref[pl.ds(t, 1), :] = jnp.sum(
            hfn * wof, axis=0, keepdims=True).astype(jnp.float32)
        pb_ref[pl.ds(tb, 1), :] = jnp.sum(
            hbn * wob, axis=0, keepdims=True).astype(jnp.float32)
        return carry

    jax.lax.fori_loop(0, S, step, 0)

    xs = xs_ref[:, :]
    m = ms_ref[:, :]
    imp = pf_ref[:, :] + pb_ref[:, :] + bout_ref[0, 0]
    out_ref[:, :] = m * xs + (1.0 - m) * imp


def _pack_weights(Wf, Uf, bf, Wb, Ub, bb, H):
    w = jnp.zeros((8 * H, _K), jnp.float32)
    UfT, UbT = Uf.T, Ub.T
    w = w.at[0:3 * H, 0:H].set(UfT)
    w = w.at[4 * H:7 * H, H:2 * H].set(UbT)
    w = w.at[0:2 * H, 2 * H].set(Wf[0, 0:2 * H])
    w = w.at[3 * H:4 * H, 2 * H].set(Wf[0, 2 * H:3 * H])
    w = w.at[4 * H:6 * H, 2 * H + 1].set(Wb[0, 0:2 * H])
    w = w.at[7 * H:8 * H, 2 * H + 1].set(Wb[0, 2 * H:3 * H])
    w = w.at[0:2 * H, 2 * H + 2].set(bf[0:2 * H])
    w = w.at[3 * H:4 * H, 2 * H + 2].set(bf[2 * H:3 * H])
    w = w.at[4 * H:6 * H, 2 * H + 2].set(bb[0:2 * H])
    w = w.at[7 * H:8 * H, 2 * H + 2].set(bb[2 * H:3 * H])
    scale = jnp.ones((8 * H, 1), jnp.float32)
    scale = scale.at[0:3 * H].set(0.5)
    scale = scale.at[4 * H:7 * H].set(0.5)
    return (w * scale).astype(jnp.bfloat16)


def kernel(x, mask, Wf, Uf, bf, Wb, Ub, bb, Wout, bout):
    B, S, N, C = x.shape
    H = Uf.shape[0]
    R = B * N
    RT = 2048
    G = R // RT

    xs = x.transpose(1, 0, 2, 3).reshape(S, R)
    ms = mask.astype(jnp.float32).transpose(1, 0, 2, 3).reshape(S, R)

    w = _pack_weights(Wf, Uf, bf, Wb, Ub, bb, H)
    wof = Wout[:H, 0:1].astype(jnp.bfloat16)
    wob = Wout[H:, 0:1].astype(jnp.bfloat16)
    bout2 = bout.reshape(1, 1)

    full = lambda shape: pl.BlockSpec(shape, lambda i: (0, 0))
    tile = pl.BlockSpec((S, RT), lambda i: (0, i))

    out = pl.pallas_call(
        _bigru_kernel,
        grid=(G,),
        in_specs=[
            tile,
            tile,
            full((8 * H, _K)),
            full((H, 1)),
            full((H, 1)),
            full((1, 1)),
        ],
        out_specs=tile,
        out_shape=jax.ShapeDtypeStruct((S, R), jnp.float32),
        scratch_shapes=[
            pltpu.VMEM((_K, RT), jnp.bfloat16),
            pltpu.VMEM((S, RT), jnp.float32),
            pltpu.VMEM((S, RT), jnp.float32),
        ],
        compiler_params=pltpu.CompilerParams(
            dimension_semantics=("arbitrary",),
        ),
    )(xs, ms, w, wof, wob, bout2)

    return out.reshape(S, B, N, C).transpose(1, 0, 2, 3)
